# baseline (speedup 1.0000x reference)
import functools
import math

import jax
import jax.numpy as jnp
from jax import lax
from jax.experimental import pallas as pl
from jax.experimental.pallas import tpu as pltpu
from jax.experimental.pallas import tpu_sc as plsc

DIM_ = 128
HALF_ = 64
BASE_ = 10000.0
CACHE_ = 4096

NC_ = 2
NS_ = 16
NW_ = NC_ * NS_


def _round_to_f16(v):
    i = jax.lax.bitcast_convert_type(v, jnp.int32)
    bias = 0x0FFF + ((i >> 13) & 1)
    r = (i + bias) & jnp.int32(~0x1FFF)
    return jax.lax.bitcast_convert_type(r, jnp.float32)


def _inv_freq_row():
    i = lax.broadcasted_iota(jnp.int32, (1, HALF_), 1).astype(jnp.float32)
    return jnp.exp(i * (-math.log(BASE_) / HALF_))


def _sin_table_body(sin_ref):
    p = lax.broadcasted_iota(jnp.int32, (CACHE_, HALF_), 0).astype(jnp.float32)
    freqs = p * _inv_freq_row()
    s = _round_to_f16(jnp.sin(freqs))
    sin_ref[...] = jnp.concatenate([s, s], axis=-1)


def _sc_lookup(sin_t, pos_flat, B, H, S, D):
    chunk = (B * S) // NW_
    mesh = plsc.VectorSubcoreMesh(
        core_axis_name="c", subcore_axis_name="s",
        num_cores=NC_, num_subcores=NS_)
    out_t = jax.ShapeDtypeStruct((B, H, S, D), jnp.float32)

    @functools.partial(
        pl.kernel,
        out_type=out_t,
        mesh=mesh,
        cost_estimate=pl.CostEstimate(
            flops=0, transcendentals=0,
            bytes_accessed=(B * H * S * D + B * S * D) * 4 + B * S * 4),
        scratch_types=[
            pltpu.VMEM((chunk,), jnp.int32),
            pltpu.VMEM((chunk, D), jnp.float32),
            pltpu.SemaphoreType.DMA,
            pltpu.SemaphoreType.DMA,
        ],
    )
    def sc_kernel(sin_t_hbm, pos_hbm, sin_out, idx_v, sin_v, gsem, wsem):
        wid = lax.axis_index("s") * NC_ + lax.axis_index("c")
        base = wid * chunk
        b = base // S
        s0 = base % S
        pltpu.sync_copy(pos_hbm.at[pl.ds(base, chunk)], idx_v)
        pltpu.async_copy(sin_t_hbm.at[idx_v], sin_v, gsem).wait()
        grp = 8
        for g in range(0, H, grp):
            cps = []
            for h in range(g, g + grp):
                cps.append(pltpu.async_copy(
                    sin_v, sin_out.at[b, h, pl.ds(s0, chunk)], wsem))
            for cp in cps:
                cp.wait()

    return sc_kernel(sin_t, pos_flat)


def _cos_compact_body(pos_ref, cos_ref):
    pos = pos_ref[0, :, :].astype(jnp.float32)
    freqs = pos * _inv_freq_row()
    c = _round_to_f16(jnp.cos(freqs))
    cos_ref[0, :, :] = jnp.concatenate([c, c], axis=-1)


def _bcast_body(cos_c_ref, cos_ref):
    cos_ref[...] = jnp.broadcast_to(cos_c_ref[...][None, :, :, :], cos_ref.shape)


def _tc_cos(position_ids, B, H, S, D, interpret):
    bs2 = 2048
    nS2 = S // bs2
    pos3 = position_ids.reshape(B * nS2, bs2, 1)
    cos_c = pl.pallas_call(
        _cos_compact_body,
        grid=(B * nS2,),
        in_specs=[pl.BlockSpec((1, bs2, 1), lambda i: (i, 0, 0))],
        out_specs=pl.BlockSpec((1, bs2, D), lambda i: (i // nS2, i % nS2, 0)),
        out_shape=jax.ShapeDtypeStruct((B, S, D), jnp.float32),
        interpret=interpret,
    )(pos3)

    bs = 1024
    hg = 2
    nS = S // bs
    cos = pl.pallas_call(
        _bcast_body,
        grid=(B * nS, H // hg),
        in_specs=[pl.BlockSpec((1, bs, D), lambda i, h: (i // nS, i % nS, 0))],
        out_specs=pl.BlockSpec((1, hg, bs, D),
                               lambda i, h: (i // nS, h, i % nS, 0)),
        out_shape=jax.ShapeDtypeStruct((B, H, S, D), jnp.float32),
        interpret=interpret,
    )(cos_c)
    return cos


@functools.partial(jax.jit, static_argnames=("interpret",))
def kernel(x, position_ids, interpret=False):
    B, H, S, D = x.shape
    sin_t = pl.pallas_call(
        _sin_table_body,
        out_shape=jax.ShapeDtypeStruct((CACHE_, DIM_), jnp.float32),
        interpret=interpret,
    )()
    sin = _sc_lookup(sin_t, position_ids.reshape(B * S), B, H, S, D)
    cos = _tc_cos(position_ids, B, H, S, D, interpret)
    return cos, sin


# speedup vs baseline: 1.2903x; 1.2903x over previous
import functools
import math

import jax
import jax.numpy as jnp
from jax import lax
from jax.experimental import pallas as pl
from jax.experimental.pallas import tpu as pltpu
from jax.experimental.pallas import tpu_sc as plsc

DIM_ = 128
HALF_ = 64
BASE_ = 10000.0
CACHE_ = 4096

NC_ = 2
NS_ = 16
NW_ = NC_ * NS_


def _round_to_f16(v):
    i = jax.lax.bitcast_convert_type(v, jnp.int32)
    bias = 0x0FFF + ((i >> 13) & 1)
    r = (i + bias) & jnp.int32(~0x1FFF)
    return jax.lax.bitcast_convert_type(r, jnp.float32)


def _table_body(cos_ref, sin_ref):
    p = lax.broadcasted_iota(jnp.int32, (CACHE_, HALF_), 0).astype(jnp.float32)
    i = lax.broadcasted_iota(jnp.int32, (CACHE_, HALF_), 1).astype(jnp.float32)
    inv_freq = jnp.exp(i * (-math.log(BASE_) / HALF_))
    freqs = p * inv_freq
    c = _round_to_f16(jnp.cos(freqs))
    s = _round_to_f16(jnp.sin(freqs))
    cos_ref[...] = jnp.concatenate([c, c], axis=-1)
    sin_ref[...] = jnp.concatenate([s, s], axis=-1)


def _sc_lookup(cos_t, sin_t, pos_flat, B, H, S, D):
    chunk = (B * S) // NW_
    mesh = plsc.VectorSubcoreMesh(
        core_axis_name="c", subcore_axis_name="s",
        num_cores=NC_, num_subcores=NS_)
    out_t = jax.ShapeDtypeStruct((B, H, S, D), jnp.float32)

    @functools.partial(
        pl.kernel,
        out_type=[out_t, out_t],
        mesh=mesh,
        scratch_types=[
            pltpu.VMEM((chunk,), jnp.int32),
            pltpu.VMEM((chunk, D), jnp.float32),
            pltpu.VMEM((chunk, D), jnp.float32),
            pltpu.SemaphoreType.DMA,
            pltpu.SemaphoreType.DMA,
        ],
    )
    def sc_kernel(cos_t_hbm, sin_t_hbm, pos_hbm, cos_out, sin_out,
                  idx_v, cos_v, sin_v, gsem, wsem):
        wid = lax.axis_index("s") * NC_ + lax.axis_index("c")
        base = wid * chunk
        b = base // S
        s0 = base % S
        pltpu.sync_copy(pos_hbm.at[pl.ds(base, chunk)], idx_v)
        g1 = pltpu.async_copy(cos_t_hbm.at[idx_v], cos_v, gsem)
        g2 = pltpu.async_copy(sin_t_hbm.at[idx_v], sin_v, gsem)
        g1.wait()
        g2.wait()
        cps = []
        for h in range(H):
            cps.append(pltpu.async_copy(
                cos_v, cos_out.at[b, h, pl.ds(s0, chunk)], wsem))
            cps.append(pltpu.async_copy(
                sin_v, sin_out.at[b, h, pl.ds(s0, chunk)], wsem))
        for cp in cps:
            cp.wait()

    return sc_kernel(cos_t, sin_t, pos_flat)


@functools.partial(jax.jit, static_argnames=("interpret",))
def kernel(x, position_ids, interpret=False):
    B, H, S, D = x.shape
    cos_t, sin_t = pl.pallas_call(
        _table_body,
        out_shape=[jax.ShapeDtypeStruct((CACHE_, DIM_), jnp.float32)] * 2,
        interpret=interpret,
    )()
    cos, sin = _sc_lookup(cos_t, sin_t, position_ids.reshape(B * S),
                          B, H, S, D)
    return cos, sin


# speedup vs baseline: 1.3456x; 1.0429x over previous
import functools
import math

import jax
import jax.numpy as jnp
from jax import lax
from jax.experimental import pallas as pl
from jax.experimental.pallas import tpu as pltpu
from jax.experimental.pallas import tpu_sc as plsc

DIM_ = 128
HALF_ = 64
BASE_ = 10000.0
CACHE_ = 4096

NC_ = 2
NS_ = 16
NW_ = NC_ * NS_


def _round_to_f16(v):
    i = jax.lax.bitcast_convert_type(v, jnp.int32)
    bias = 0x0FFF + ((i >> 13) & 1)
    r = (i + bias) & jnp.int32(~0x1FFF)
    return jax.lax.bitcast_convert_type(r, jnp.float32)


def _table_body(cos_ref, sin_ref):
    nq = CACHE_ // 16
    iq = lax.broadcasted_iota(jnp.int32, (nq, HALF_), 0).astype(jnp.float32)
    ir = lax.broadcasted_iota(jnp.int32, (16, HALF_), 0).astype(jnp.float32)
    jq = lax.broadcasted_iota(jnp.int32, (nq, HALF_), 1).astype(jnp.float32)
    jr = lax.broadcasted_iota(jnp.int32, (16, HALF_), 1).astype(jnp.float32)
    c = -math.log(BASE_) / HALF_
    aq = (16.0 * iq) * jnp.exp(jq * c)
    ar = ir * jnp.exp(jr * c)
    ac, as_ = jnp.cos(aq), jnp.sin(aq)
    bc, bs = jnp.cos(ar), jnp.sin(ar)
    ace = jnp.broadcast_to(ac[:, None, :], (nq, 16, HALF_)).reshape(CACHE_, HALF_)
    ase = jnp.broadcast_to(as_[:, None, :], (nq, 16, HALF_)).reshape(CACHE_, HALF_)
    bce = jnp.broadcast_to(bc[None, :, :], (nq, 16, HALF_)).reshape(CACHE_, HALF_)
    bse = jnp.broadcast_to(bs[None, :, :], (nq, 16, HALF_)).reshape(CACHE_, HALF_)
    cv = _round_to_f16(ace * bce - ase * bse)
    sv = _round_to_f16(ase * bce + ace * bse)
    cos_ref[...] = jnp.concatenate([cv, cv], axis=-1)
    sin_ref[...] = jnp.concatenate([sv, sv], axis=-1)


def _sc_lookup(cos_t, sin_t, pos_flat, B, H, S, D):
    chunk = (B * S) // NW_
    mesh = plsc.VectorSubcoreMesh(
        core_axis_name="c", subcore_axis_name="s",
        num_cores=NC_, num_subcores=NS_)
    out_t = jax.ShapeDtypeStruct((B, H, S, D), jnp.float32)

    @functools.partial(
        pl.kernel,
        out_type=[out_t, out_t],
        mesh=mesh,
        scratch_types=[
            pltpu.VMEM((chunk,), jnp.int32),
            pltpu.VMEM((chunk, D), jnp.float32),
            pltpu.VMEM((chunk, D), jnp.float32),
            pltpu.SemaphoreType.DMA,
            pltpu.SemaphoreType.DMA,
        ],
    )
    def sc_kernel(cos_t_hbm, sin_t_hbm, pos_hbm, cos_out, sin_out,
                  idx_v, cos_v, sin_v, gsem, wsem):
        wid = lax.axis_index("s") * NC_ + lax.axis_index("c")
        base = wid * chunk
        b = base // S
        s0 = base % S
        pltpu.sync_copy(pos_hbm.at[pl.ds(base, chunk)], idx_v)
        g1 = pltpu.async_copy(cos_t_hbm.at[idx_v], cos_v, gsem)
        g2 = pltpu.async_copy(sin_t_hbm.at[idx_v], sin_v, gsem)
        g1.wait()
        g2.wait()
        cps = []
        for h in range(H):
            cps.append(pltpu.async_copy(
                cos_v, cos_out.at[b, h, pl.ds(s0, chunk)], wsem))
            cps.append(pltpu.async_copy(
                sin_v, sin_out.at[b, h, pl.ds(s0, chunk)], wsem))
        for cp in cps:
            cp.wait()

    return sc_kernel(cos_t, sin_t, pos_flat)


@functools.partial(jax.jit, static_argnames=("interpret",))
def kernel(x, position_ids, interpret=False):
    B, H, S, D = x.shape
    cos_t, sin_t = pl.pallas_call(
        _table_body,
        out_shape=[jax.ShapeDtypeStruct((CACHE_, DIM_), jnp.float32)] * 2,
        interpret=interpret,
    )()
    cos, sin = _sc_lookup(cos_t, sin_t, position_ids.reshape(B * S),
                          B, H, S, D)
    return cos, sin


# speedup vs baseline: 1.3596x; 1.0104x over previous
import functools
import math

import jax
import jax.numpy as jnp
from jax import lax
from jax.experimental import pallas as pl
from jax.experimental.pallas import tpu as pltpu
from jax.experimental.pallas import tpu_sc as plsc

DIM_ = 128
HALF_ = 64
BASE_ = 10000.0
CACHE_ = 4096

NC_ = 2
NS_ = 16
NW_ = NC_ * NS_


def _round_to_f16(v):
    i = jax.lax.bitcast_convert_type(v, jnp.int32)
    bias = 0x0FFF + ((i >> 13) & 1)
    r = (i + bias) & jnp.int32(~0x1FFF)
    return jax.lax.bitcast_convert_type(r, jnp.float32)


def _table_body(cos_ref, sin_ref):
    nq = CACHE_ // 16
    iq = lax.broadcasted_iota(jnp.int32, (nq, HALF_), 0).astype(jnp.float32)
    ir = lax.broadcasted_iota(jnp.int32, (16, HALF_), 0).astype(jnp.float32)
    jq = lax.broadcasted_iota(jnp.int32, (nq, HALF_), 1).astype(jnp.float32)
    jr = lax.broadcasted_iota(jnp.int32, (16, HALF_), 1).astype(jnp.float32)
    c = -math.log(BASE_) / HALF_
    aq = (16.0 * iq) * jnp.exp(jq * c)
    ar = ir * jnp.exp(jr * c)
    ac, as_ = jnp.cos(aq), jnp.sin(aq)
    bc, bs = jnp.cos(ar), jnp.sin(ar)
    ace = jnp.broadcast_to(ac[:, None, :], (nq, 16, HALF_)).reshape(CACHE_, HALF_)
    ase = jnp.broadcast_to(as_[:, None, :], (nq, 16, HALF_)).reshape(CACHE_, HALF_)
    bce = jnp.broadcast_to(bc[None, :, :], (nq, 16, HALF_)).reshape(CACHE_, HALF_)
    bse = jnp.broadcast_to(bs[None, :, :], (nq, 16, HALF_)).reshape(CACHE_, HALF_)
    cv = _round_to_f16(ace * bce - ase * bse)
    sv = _round_to_f16(ase * bce + ace * bse)
    cos_ref[...] = jnp.concatenate([cv, cv], axis=-1)
    sin_ref[...] = jnp.concatenate([sv, sv], axis=-1)


def _sc_lookup(cos_t, sin_t, pos_flat, B, H, S, D):
    chunk = (B * S) // NW_
    mesh = plsc.VectorSubcoreMesh(
        core_axis_name="c", subcore_axis_name="s",
        num_cores=NC_, num_subcores=NS_)
    out_t = jax.ShapeDtypeStruct((B, H, S, D), jnp.float32)

    @functools.partial(
        pl.kernel,
        out_type=[out_t, out_t],
        mesh=mesh,
        scratch_types=[
            pltpu.VMEM((chunk,), jnp.int32),
            pltpu.VMEM((chunk, D), jnp.float32),
            pltpu.VMEM((chunk, D), jnp.float32),
            pltpu.SemaphoreType.DMA,
            pltpu.SemaphoreType.DMA,
        ],
    )
    def sc_kernel(cos_t_hbm, sin_t_hbm, pos_hbm, cos_out, sin_out,
                  idx_v, cos_v, sin_v, gsem, wsem):
        wid = lax.axis_index("s") * NC_ + lax.axis_index("c")
        base = wid * chunk
        b = base // S
        s0 = base % S
        pltpu.sync_copy(pos_hbm.at[pl.ds(base, chunk)], idx_v)
        g1 = pltpu.async_copy(cos_t_hbm.at[idx_v], cos_v, gsem)
        g2 = pltpu.async_copy(sin_t_hbm.at[idx_v], sin_v, gsem)
        cps = []
        g1.wait()
        for h in range(H):
            cps.append(pltpu.async_copy(
                cos_v, cos_out.at[b, h, pl.ds(s0, chunk)], wsem))
        g2.wait()
        for h in range(H):
            cps.append(pltpu.async_copy(
                sin_v, sin_out.at[b, h, pl.ds(s0, chunk)], wsem))
        for cp in cps:
            cp.wait()

    return sc_kernel(cos_t, sin_t, pos_flat)


@functools.partial(jax.jit, static_argnames=("interpret",))
def kernel(x, position_ids, interpret=False):
    B, H, S, D = x.shape
    cos_t, sin_t = pl.pallas_call(
        _table_body,
        out_shape=[jax.ShapeDtypeStruct((CACHE_, DIM_), jnp.float32)] * 2,
        interpret=interpret,
    )()
    cos, sin = _sc_lookup(cos_t, sin_t, position_ids.reshape(B * S),
                          B, H, S, D)
    return cos, sin
